# Initial kernel scaffold; baseline (speedup 1.0000x reference)
import jax
import jax.numpy as jnp
from jax.experimental import pallas as pl


def kernel(x, nw0, nw1, nw2, nw3, W1, W2, W3):
    raise NotImplementedError("write your pallas kernel here")


# speedup vs baseline: 1.4317x; 1.4317x over previous
import functools

import jax
import jax.numpy as jnp
from jax.experimental import pallas as pl
from jax.experimental.pallas import tpu as pltpu

H = 4096
N_TOK = 8192
EPS = 1e-6
FP8_MAX = 448.0

WB = 256
NB = H // WB
BM = 256


def _wmax_body(w1_ref, w2_ref, w3_ref, m1_ref, m2_ref, m3_ref):
    m1_ref[...] = jnp.max(jnp.abs(w1_ref[...])).reshape(1, 1, 1)
    m2_ref[...] = jnp.max(jnp.abs(w2_ref[...])).reshape(1, 1, 1)
    m3_ref[...] = jnp.max(jnp.abs(w3_ref[...])).reshape(1, 1, 1)


def _wmax(W1, W2, W3):
    spec = pl.BlockSpec((WB, H), lambda i: (i, 0))
    ospec = pl.BlockSpec((1, 1, 1), lambda i: (i, 0, 0))
    return pl.pallas_call(
        _wmax_body,
        grid=(NB,),
        in_specs=[spec, spec, spec],
        out_specs=[ospec, ospec, ospec],
        out_shape=[jax.ShapeDtypeStruct((NB, 1, 1), jnp.float32)] * 3,
        compiler_params=pltpu.CompilerParams(
            dimension_semantics=("parallel",),
        ),
        name="wmax",
    )(W1, W2, W3)


def _wcast_body(w1_ref, w2_ref, w3_ref, m1_ref, m2_ref, m3_ref,
                q1_ref, q2_ref, q3_ref, s1_ref, s2_ref, s3_ref):
    for w_ref, m_ref, q_ref, s_ref in (
        (w1_ref, m1_ref, q1_ref, s1_ref),
        (w2_ref, m2_ref, q2_ref, s2_ref),
        (w3_ref, m3_ref, q3_ref, s3_ref),
    ):
        amax = jnp.max(m_ref[...])
        scale = jnp.maximum(amax / FP8_MAX, 1e-12)
        q_ref[...] = jnp.clip(w_ref[...] / scale, -FP8_MAX, FP8_MAX).astype(
            jnp.float8_e4m3fn)
        s_ref[...] = scale.reshape(1, 1)


def _wcast(W1, W2, W3, m1, m2, m3):
    wspec = pl.BlockSpec((WB, H), lambda i: (i, 0))
    mspec = pl.BlockSpec((NB, 1, 1), lambda i: (0, 0, 0))
    qspec = pl.BlockSpec((WB, H), lambda i: (i, 0))
    sspec = pl.BlockSpec((1, 1), lambda i: (0, 0))
    return pl.pallas_call(
        _wcast_body,
        grid=(NB,),
        in_specs=[wspec, wspec, wspec, mspec, mspec, mspec],
        out_specs=[qspec, qspec, qspec, sspec, sspec, sspec],
        out_shape=[jax.ShapeDtypeStruct((H, H), jnp.float8_e4m3fn)] * 3
        + [jax.ShapeDtypeStruct((1, 1), jnp.float32)] * 3,
        compiler_params=pltpu.CompilerParams(
            dimension_semantics=("parallel",),
        ),
        name="wcast",
    )(W1, W2, W3, m1, m2, m3)


def _layer_body(do_relu, do_final_norm, *refs):
    if do_final_norm:
        resid_ref, nw_ref, qw_ref, sw_ref, nwf_ref, out_ref = refs
    else:
        resid_ref, nw_ref, qw_ref, sw_ref, out_ref = refs
    r = resid_ref[...]
    if do_relu:
        r = jnp.maximum(r, 0.0)
    var = jnp.mean(r * r, axis=-1, keepdims=True)
    y = r * jax.lax.rsqrt(var + EPS) * nw_ref[...]
    amax = jnp.max(jnp.abs(y), axis=-1, keepdims=True)
    s = jnp.maximum(amax / FP8_MAX, 1e-12)
    q = jnp.clip(y / s, -FP8_MAX, FP8_MAX).astype(jnp.float8_e4m3fn)
    acc = jax.lax.dot_general(
        q, qw_ref[...], (((1,), (1,)), ((), ())),
        preferred_element_type=jnp.float32)
    new_resid = acc * (s * sw_ref[0, 0]) + r
    if do_final_norm:
        var2 = jnp.mean(new_resid * new_resid, axis=-1, keepdims=True)
        out_ref[...] = new_resid * jax.lax.rsqrt(var2 + EPS) * nwf_ref[...]
    else:
        out_ref[...] = new_resid


def _layer(resid, nw, qw, sw, nwf=None, do_relu=False):
    in_specs = [
        pl.BlockSpec((BM, H), lambda i: (i, 0)),
        pl.BlockSpec((1, H), lambda i: (0, 0)),
        pl.BlockSpec((H, H), lambda i: (0, 0)),
        pl.BlockSpec(memory_space=pltpu.SMEM),
    ]
    args = [resid, nw.reshape(1, H), qw, sw]
    if nwf is not None:
        in_specs.append(pl.BlockSpec((1, H), lambda i: (0, 0)))
        args.append(nwf.reshape(1, H))
    body = functools.partial(_layer_body, do_relu, nwf is not None)
    return pl.pallas_call(
        body,
        grid=(N_TOK // BM,),
        in_specs=in_specs,
        out_specs=pl.BlockSpec((BM, H), lambda i: (i, 0)),
        out_shape=jax.ShapeDtypeStruct((N_TOK, H), jnp.float32),
        compiler_params=pltpu.CompilerParams(
            dimension_semantics=("parallel",),
            vmem_limit_bytes=56 * 1024 * 1024,
        ),
        name="fused_layer",
    )(*args)


def kernel(x, nw0, nw1, nw2, nw3, W1, W2, W3):
    m1, m2, m3 = _wmax(W1, W2, W3)
    qw1, qw2, qw3, s1, s2, s3 = _wcast(W1, W2, W3, m1, m2, m3)
    h1 = _layer(x, nw0, qw1, s1, do_relu=True)
    h2 = _layer(h1, nw1, qw2, s2)
    return _layer(h2, nw2, qw3, s3, nwf=nw3)


# speedup vs baseline: 1.4319x; 1.0002x over previous
import functools

import jax
import jax.numpy as jnp
from jax.experimental import pallas as pl
from jax.experimental.pallas import tpu as pltpu

H = 4096
N_TOK = 8192
EPS = 1e-6
FP8_MAX = 448.0

WB = 256
NB = H // WB
BM = 256


def _wmax_body(w1_ref, w2_ref, w3_ref, m1_ref, m2_ref, m3_ref):
    m1_ref[...] = jnp.max(jnp.abs(w1_ref[...])).reshape(1, 1, 1)
    m2_ref[...] = jnp.max(jnp.abs(w2_ref[...])).reshape(1, 1, 1)
    m3_ref[...] = jnp.max(jnp.abs(w3_ref[...])).reshape(1, 1, 1)


def _wmax(W1, W2, W3):
    spec = pl.BlockSpec((WB, H), lambda i: (i, 0))
    ospec = pl.BlockSpec((1, 1, 1), lambda i: (i, 0, 0))
    return pl.pallas_call(
        _wmax_body,
        grid=(NB,),
        in_specs=[spec, spec, spec],
        out_specs=[ospec, ospec, ospec],
        out_shape=[jax.ShapeDtypeStruct((NB, 1, 1), jnp.float32)] * 3,
        compiler_params=pltpu.CompilerParams(
            dimension_semantics=("parallel",),
        ),
        name="wmax",
    )(W1, W2, W3)


def _wcast_body(w1_ref, w2_ref, w3_ref, m1_ref, m2_ref, m3_ref,
                q1_ref, q2_ref, q3_ref, s1_ref, s2_ref, s3_ref):
    for w_ref, m_ref, q_ref, s_ref in (
        (w1_ref, m1_ref, q1_ref, s1_ref),
        (w2_ref, m2_ref, q2_ref, s2_ref),
        (w3_ref, m3_ref, q3_ref, s3_ref),
    ):
        amax = jnp.max(m_ref[...])
        scale = jnp.maximum(amax / FP8_MAX, 1e-12)
        inv = 1.0 / scale
        q_ref[...] = jnp.clip(w_ref[...] * inv, -FP8_MAX, FP8_MAX).astype(
            jnp.float8_e4m3fn)
        s_ref[...] = scale.reshape(1, 1)


def _wcast(W1, W2, W3, m1, m2, m3):
    wspec = pl.BlockSpec((WB, H), lambda i: (i, 0))
    mspec = pl.BlockSpec((NB, 1, 1), lambda i: (0, 0, 0))
    qspec = pl.BlockSpec((WB, H), lambda i: (i, 0))
    sspec = pl.BlockSpec((1, 1), lambda i: (0, 0))
    return pl.pallas_call(
        _wcast_body,
        grid=(NB,),
        in_specs=[wspec, wspec, wspec, mspec, mspec, mspec],
        out_specs=[qspec, qspec, qspec, sspec, sspec, sspec],
        out_shape=[jax.ShapeDtypeStruct((H, H), jnp.float8_e4m3fn)] * 3
        + [jax.ShapeDtypeStruct((1, 1), jnp.float32)] * 3,
        compiler_params=pltpu.CompilerParams(
            dimension_semantics=("parallel",),
        ),
        name="wcast",
    )(W1, W2, W3, m1, m2, m3)


def _layer_body(do_relu, do_final_norm, *refs):
    if do_final_norm:
        resid_ref, nw_ref, qw_ref, sw_ref, nwf_ref, out_ref = refs
    else:
        resid_ref, nw_ref, qw_ref, sw_ref, out_ref = refs
    r = resid_ref[...]
    if do_relu:
        r = jnp.maximum(r, 0.0)
    var = jnp.mean(r * r, axis=-1, keepdims=True)
    y = r * jax.lax.rsqrt(var + EPS) * nw_ref[...]
    amax = jnp.max(jnp.abs(y), axis=-1, keepdims=True)
    s = jnp.maximum(amax / FP8_MAX, 1e-12)
    q = jnp.clip(y * (1.0 / s), -FP8_MAX, FP8_MAX).astype(jnp.float8_e4m3fn)
    acc = jax.lax.dot_general(
        q, qw_ref[...], (((1,), (1,)), ((), ())),
        preferred_element_type=jnp.float32)
    new_resid = acc * (s * sw_ref[0, 0]) + r
    if do_final_norm:
        var2 = jnp.mean(new_resid * new_resid, axis=-1, keepdims=True)
        out_ref[...] = new_resid * jax.lax.rsqrt(var2 + EPS) * nwf_ref[...]
    else:
        out_ref[...] = new_resid


def _layer(resid, nw, qw, sw, nwf=None, do_relu=False):
    in_specs = [
        pl.BlockSpec((BM, H), lambda i: (i, 0)),
        pl.BlockSpec((1, H), lambda i: (0, 0)),
        pl.BlockSpec((H, H), lambda i: (0, 0)),
        pl.BlockSpec(memory_space=pltpu.SMEM),
    ]
    args = [resid, nw.reshape(1, H), qw, sw]
    if nwf is not None:
        in_specs.append(pl.BlockSpec((1, H), lambda i: (0, 0)))
        args.append(nwf.reshape(1, H))
    body = functools.partial(_layer_body, do_relu, nwf is not None)
    return pl.pallas_call(
        body,
        grid=(N_TOK // BM,),
        in_specs=in_specs,
        out_specs=pl.BlockSpec((BM, H), lambda i: (i, 0)),
        out_shape=jax.ShapeDtypeStruct((N_TOK, H), jnp.float32),
        compiler_params=pltpu.CompilerParams(
            dimension_semantics=("parallel",),
            vmem_limit_bytes=56 * 1024 * 1024,
        ),
        name="fused_layer",
    )(*args)


def kernel(x, nw0, nw1, nw2, nw3, W1, W2, W3):
    m1, m2, m3 = _wmax(W1, W2, W3)
    qw1, qw2, qw3, s1, s2, s3 = _wcast(W1, W2, W3, m1, m2, m3)
    h1 = _layer(x, nw0, qw1, s1, do_relu=True)
    h2 = _layer(h1, nw1, qw2, s2)
    return _layer(h2, nw2, qw3, s3, nwf=nw3)


# speedup vs baseline: 1.4529x; 1.0147x over previous
import functools

import jax
import jax.numpy as jnp
from jax.experimental import pallas as pl
from jax.experimental.pallas import tpu as pltpu

H = 4096
N_TOK = 8192
EPS = 1e-6
FP8_MAX = 448.0

WB = 256
NB = H // WB
BM = 256


def _wmax_body(w1_ref, w2_ref, w3_ref, m1_ref, m2_ref, m3_ref):
    m1_ref[...] = jnp.max(jnp.abs(w1_ref[...])).reshape(1, 1, 1)
    m2_ref[...] = jnp.max(jnp.abs(w2_ref[...])).reshape(1, 1, 1)
    m3_ref[...] = jnp.max(jnp.abs(w3_ref[...])).reshape(1, 1, 1)


def _wmax(W1, W2, W3):
    spec = pl.BlockSpec((WB, H), lambda i: (i, 0))
    ospec = pl.BlockSpec((1, 1, 1), lambda i: (i, 0, 0))
    return pl.pallas_call(
        _wmax_body,
        grid=(NB,),
        in_specs=[spec, spec, spec],
        out_specs=[ospec, ospec, ospec],
        out_shape=[jax.ShapeDtypeStruct((NB, 1, 1), jnp.float32)] * 3,
        compiler_params=pltpu.CompilerParams(
            dimension_semantics=("parallel",),
        ),
        name="wmax",
    )(W1, W2, W3)


def _wcast_body(w1_ref, w2_ref, w3_ref, m1_ref, m2_ref, m3_ref,
                q1_ref, q2_ref, q3_ref, s1_ref, s2_ref, s3_ref):
    for w_ref, m_ref, q_ref, s_ref in (
        (w1_ref, m1_ref, q1_ref, s1_ref),
        (w2_ref, m2_ref, q2_ref, s2_ref),
        (w3_ref, m3_ref, q3_ref, s3_ref),
    ):
        amax = jnp.max(m_ref[...])
        scale = jnp.maximum(amax / FP8_MAX, 1e-12)
        inv = 1.0 / scale
        q_ref[...] = jnp.clip(w_ref[...] * inv, -FP8_MAX, FP8_MAX).astype(
            jnp.float8_e4m3fn)
        s_ref[...] = scale.reshape(1, 1)


def _wcast(W1, W2, W3, m1, m2, m3):
    wspec = pl.BlockSpec((WB, H), lambda i: (i, 0))
    mspec = pl.BlockSpec((NB, 1, 1), lambda i: (0, 0, 0))
    qspec = pl.BlockSpec((WB, H), lambda i: (i, 0))
    sspec = pl.BlockSpec((1, 1), lambda i: (0, 0))
    return pl.pallas_call(
        _wcast_body,
        grid=(NB,),
        in_specs=[wspec, wspec, wspec, mspec, mspec, mspec],
        out_specs=[qspec, qspec, qspec, sspec, sspec, sspec],
        out_shape=[jax.ShapeDtypeStruct((H, H), jnp.float8_e4m3fn)] * 3
        + [jax.ShapeDtypeStruct((1, 1), jnp.float32)] * 3,
        compiler_params=pltpu.CompilerParams(
            dimension_semantics=("parallel",),
        ),
        name="wcast",
    )(W1, W2, W3, m1, m2, m3)


def _layer_body(do_relu, do_final_norm, *refs):
    if do_final_norm:
        resid_ref, nw_ref, qw_ref, sw_ref, nwf_ref, out_ref = refs
    else:
        resid_ref, nw_ref, qw_ref, sw_ref, out_ref = refs
    r = resid_ref[...]
    if do_relu:
        r = jnp.maximum(r, 0.0)
    var = jnp.mean(r * r, axis=-1, keepdims=True)
    y = r * jax.lax.rsqrt(var + EPS) * nw_ref[...]
    amax = jnp.max(jnp.abs(y), axis=-1, keepdims=True)
    s = jnp.maximum(amax / FP8_MAX, 1e-12)
    q = jnp.clip(y * (1.0 / s), -FP8_MAX, FP8_MAX).astype(jnp.float8_e4m3fn)
    acc = jax.lax.dot_general(
        q, qw_ref[...], (((1,), (1,)), ((), ())),
        preferred_element_type=jnp.float32)
    new_resid = acc * (s * sw_ref[0, 0]) + r
    if do_final_norm:
        var2 = jnp.mean(new_resid * new_resid, axis=-1, keepdims=True)
        out_ref[...] = new_resid * jax.lax.rsqrt(var2 + EPS) * nwf_ref[...]
    else:
        out_ref[...] = new_resid


def _layer(resid, nw, qw, sw, nwf=None, do_relu=False):
    in_specs = [
        pl.BlockSpec((BM, H), lambda i: (i, 0)),
        pl.BlockSpec((1, H), lambda i: (0, 0)),
        pl.BlockSpec((H, H), lambda i: (0, 0)),
        pl.BlockSpec(memory_space=pltpu.SMEM),
    ]
    args = [resid, nw.reshape(1, H), qw, sw]
    if nwf is not None:
        in_specs.append(pl.BlockSpec((1, H), lambda i: (0, 0)))
        args.append(nwf.reshape(1, H))
    body = functools.partial(_layer_body, do_relu, nwf is not None)
    return pl.pallas_call(
        body,
        grid=(N_TOK // BM,),
        in_specs=in_specs,
        out_specs=pl.BlockSpec((BM, H), lambda i: (i, 0)),
        out_shape=jax.ShapeDtypeStruct((N_TOK, H), jnp.float32),
        compiler_params=pltpu.CompilerParams(
            dimension_semantics=("parallel",),
            vmem_limit_bytes=56 * 1024 * 1024,
        ),
        name="fused_layer",
    )(*args)


FBM = 256


def _fused2_body(x_ref, nw0_ref, nw1_ref, qw1_ref, qw2_ref, s_ref, out_ref):
    r = jnp.maximum(x_ref[...], 0.0)
    for li, (nw_ref, qw_ref) in enumerate(
            ((nw0_ref, qw1_ref), (nw1_ref, qw2_ref))):
        var = jnp.mean(r * r, axis=-1, keepdims=True)
        rs = jax.lax.rsqrt(var + EPS)
        y = r * rs * nw_ref[...]
        amax = jnp.max(jnp.abs(y), axis=-1, keepdims=True)
        s = jnp.maximum(amax / FP8_MAX, 1e-12)
        q = jnp.clip(y * (1.0 / s), -FP8_MAX, FP8_MAX).astype(
            jnp.float8_e4m3fn)
        acc = jax.lax.dot_general(
            q, qw_ref[...], (((1,), (1,)), ((), ())),
            preferred_element_type=jnp.float32)
        r = acc * (s * s_ref[0, li]) + r
    out_ref[...] = r


def _fused2(x, nw0, nw1, qw1, qw2, sws):
    vspec = pl.BlockSpec((1, H), lambda i: (0, 0))
    wspec = pl.BlockSpec((H, H), lambda i: (0, 0))
    return pl.pallas_call(
        _fused2_body,
        grid=(N_TOK // FBM,),
        in_specs=[
            pl.BlockSpec((FBM, H), lambda i: (i, 0)),
            vspec, vspec,
            wspec, wspec,
            pl.BlockSpec(memory_space=pltpu.SMEM),
        ],
        out_specs=pl.BlockSpec((FBM, H), lambda i: (i, 0)),
        out_shape=jax.ShapeDtypeStruct((N_TOK, H), jnp.float32),
        compiler_params=pltpu.CompilerParams(
            dimension_semantics=("parallel",),
            vmem_limit_bytes=58 * 1024 * 1024,
        ),
        name="fused2",
    )(x, nw0.reshape(1, H), nw1.reshape(1, H), qw1, qw2, sws)


def kernel(x, nw0, nw1, nw2, nw3, W1, W2, W3):
    m1, m2, m3 = _wmax(W1, W2, W3)
    qw1, qw2, qw3, s1, s2, s3 = _wcast(W1, W2, W3, m1, m2, m3)
    sws = jnp.concatenate([s1, s2], axis=1)
    h2 = _fused2(x, nw0, nw1, qw1, qw2, sws)
    return _layer(h2, nw2, qw3, s3, nwf=nw3)


# speedup vs baseline: 1.4851x; 1.0222x over previous
import functools

import jax
import jax.numpy as jnp
from jax.experimental import pallas as pl
from jax.experimental.pallas import tpu as pltpu

H = 4096
N_TOK = 8192
EPS = 1e-6
FP8_MAX = 448.0

WB = 256
NB = H // WB
BM = 512


def _wmax_body(w1_ref, w2_ref, w3_ref, m1_ref, m2_ref, m3_ref):
    m1_ref[...] = jnp.max(jnp.abs(w1_ref[...])).reshape(1, 1, 1)
    m2_ref[...] = jnp.max(jnp.abs(w2_ref[...])).reshape(1, 1, 1)
    m3_ref[...] = jnp.max(jnp.abs(w3_ref[...])).reshape(1, 1, 1)


def _wmax(W1, W2, W3):
    spec = pl.BlockSpec((WB, H), lambda i: (i, 0))
    ospec = pl.BlockSpec((1, 1, 1), lambda i: (i, 0, 0))
    return pl.pallas_call(
        _wmax_body,
        grid=(NB,),
        in_specs=[spec, spec, spec],
        out_specs=[ospec, ospec, ospec],
        out_shape=[jax.ShapeDtypeStruct((NB, 1, 1), jnp.float32)] * 3,
        compiler_params=pltpu.CompilerParams(
            dimension_semantics=("parallel",),
        ),
        name="wmax",
    )(W1, W2, W3)


def _wcast_body(w1_ref, w2_ref, w3_ref, m1_ref, m2_ref, m3_ref,
                q1_ref, q2_ref, q3_ref, s1_ref, s2_ref, s3_ref):
    for w_ref, m_ref, q_ref, s_ref in (
        (w1_ref, m1_ref, q1_ref, s1_ref),
        (w2_ref, m2_ref, q2_ref, s2_ref),
        (w3_ref, m3_ref, q3_ref, s3_ref),
    ):
        amax = jnp.max(m_ref[...])
        scale = jnp.maximum(amax / FP8_MAX, 1e-12)
        inv = 1.0 / scale
        q_ref[...] = jnp.clip(w_ref[...] * inv, -FP8_MAX, FP8_MAX).astype(
            jnp.float8_e4m3fn)
        s_ref[...] = scale.reshape(1, 1)


def _wcast(W1, W2, W3, m1, m2, m3):
    wspec = pl.BlockSpec((WB, H), lambda i: (i, 0))
    mspec = pl.BlockSpec((NB, 1, 1), lambda i: (0, 0, 0))
    qspec = pl.BlockSpec((WB, H), lambda i: (i, 0))
    sspec = pl.BlockSpec((1, 1), lambda i: (0, 0))
    return pl.pallas_call(
        _wcast_body,
        grid=(NB,),
        in_specs=[wspec, wspec, wspec, mspec, mspec, mspec],
        out_specs=[qspec, qspec, qspec, sspec, sspec, sspec],
        out_shape=[jax.ShapeDtypeStruct((H, H), jnp.float8_e4m3fn)] * 3
        + [jax.ShapeDtypeStruct((1, 1), jnp.float32)] * 3,
        compiler_params=pltpu.CompilerParams(
            dimension_semantics=("parallel",),
        ),
        name="wcast",
    )(W1, W2, W3, m1, m2, m3)


def _layer_body(do_relu, do_final_norm, *refs):
    resid_ref, nw_ref, qw_ref, sw_ref, out_ref = refs
    r = resid_ref[...]
    if do_relu:
        r = jnp.maximum(r, 0.0)
    t = r * nw_ref[0:1, :]
    var = jnp.mean(r * r, axis=-1, keepdims=True)
    rs = jax.lax.rsqrt(var + EPS)
    amax = rs * jnp.max(jnp.abs(t), axis=-1, keepdims=True)
    s = jnp.maximum(amax / FP8_MAX, 1e-12)
    q = jnp.clip(t * (rs / s), -FP8_MAX, FP8_MAX).astype(jnp.float8_e4m3fn)
    out_ref[...] = jax.lax.dot_general(
        q, qw_ref[...], (((1,), (1,)), ((), ())),
        preferred_element_type=jnp.float32)
    new_resid = out_ref[...] * (s * sw_ref[0, 0]) + r
    if do_final_norm:
        var2 = jnp.mean(new_resid * new_resid, axis=-1, keepdims=True)
        out_ref[...] = new_resid * jax.lax.rsqrt(var2 + EPS) * nw_ref[1:2, :]
    else:
        out_ref[...] = new_resid


def _layer(resid, nw, qw, sw, nwf=None, do_relu=False):
    nwarr = (nw.reshape(1, H) if nwf is None
             else jnp.stack([nw, nwf], axis=0))
    in_specs = [
        pl.BlockSpec((BM, H), lambda i: (i, 0)),
        pl.BlockSpec(nwarr.shape, lambda i: (0, 0)),
        pl.BlockSpec((H, H), lambda i: (0, 0)),
        pl.BlockSpec(memory_space=pltpu.SMEM),
    ]
    args = [resid, nwarr, qw, sw]
    body = functools.partial(_layer_body, do_relu, nwf is not None)
    return pl.pallas_call(
        body,
        grid=(N_TOK // BM,),
        in_specs=in_specs,
        out_specs=pl.BlockSpec((BM, H), lambda i: (i, 0)),
        out_shape=jax.ShapeDtypeStruct((N_TOK, H), jnp.float32),
        compiler_params=pltpu.CompilerParams(
            dimension_semantics=("parallel",),
            vmem_limit_bytes=58 * 1024 * 1024,
        ),
        name="fused_layer",
    )(*args)


FBM = 256


def _fused2_body(x_ref, nw0_ref, nw1_ref, qw1_ref, qw2_ref, s_ref, out_ref):
    r = jnp.maximum(x_ref[...], 0.0)
    for li, (nw_ref, qw_ref) in enumerate(
            ((nw0_ref, qw1_ref), (nw1_ref, qw2_ref))):
        t = r * nw_ref[...]
        var = jnp.mean(r * r, axis=-1, keepdims=True)
        rs = jax.lax.rsqrt(var + EPS)
        amax = rs * jnp.max(jnp.abs(t), axis=-1, keepdims=True)
        s = jnp.maximum(amax / FP8_MAX, 1e-12)
        q = jnp.clip(t * (rs / s), -FP8_MAX, FP8_MAX).astype(
            jnp.float8_e4m3fn)
        acc = jax.lax.dot_general(
            q, qw_ref[...], (((1,), (1,)), ((), ())),
            preferred_element_type=jnp.float32)
        r = acc * (s * s_ref[0, li]) + r
    out_ref[...] = r


def _fused2(x, nw0, nw1, qw1, qw2, sws):
    vspec = pl.BlockSpec((1, H), lambda i: (0, 0))
    wspec = pl.BlockSpec((H, H), lambda i: (0, 0))
    return pl.pallas_call(
        _fused2_body,
        grid=(N_TOK // FBM,),
        in_specs=[
            pl.BlockSpec((FBM, H), lambda i: (i, 0)),
            vspec, vspec,
            wspec, wspec,
            pl.BlockSpec(memory_space=pltpu.SMEM),
        ],
        out_specs=pl.BlockSpec((FBM, H), lambda i: (i, 0)),
        out_shape=jax.ShapeDtypeStruct((N_TOK, H), jnp.float32),
        compiler_params=pltpu.CompilerParams(
            dimension_semantics=("parallel",),
            vmem_limit_bytes=58 * 1024 * 1024,
        ),
        name="fused2",
    )(x, nw0.reshape(1, H), nw1.reshape(1, H), qw1, qw2, sws)


def kernel(x, nw0, nw1, nw2, nw3, W1, W2, W3):
    m1, m2, m3 = _wmax(W1, W2, W3)
    qw1, qw2, qw3, s1, s2, s3 = _wcast(W1, W2, W3, m1, m2, m3)
    sws = jnp.concatenate([s1, s2], axis=1)
    h2 = _fused2(x, nw0, nw1, qw1, qw2, sws)
    return _layer(h2, nw2, qw3, s3, nwf=nw3)


# speedup vs baseline: 1.4879x; 1.0018x over previous
import functools

import jax
import jax.numpy as jnp
from jax.experimental import pallas as pl
from jax.experimental.pallas import tpu as pltpu

H = 4096
N_TOK = 8192
EPS = 1e-6
FP8_MAX = 448.0

WB = 256
NB = H // WB
BM = 512


def _wprep_body(w1_ref, w2_ref, w3_ref,
                q1_ref, q2_ref, q3_ref, s1_ref, s2_ref, s3_ref, msc_ref):
    p = pl.program_id(0)
    i = pl.program_id(1)

    @pl.when(p == 0)
    def _phase_max():
        for k, w_ref in enumerate((w1_ref, w2_ref, w3_ref)):
            m = jnp.max(jnp.abs(w_ref[...]))
            prev = jnp.where(i == 0, 0.0, msc_ref[k])
            msc_ref[k] = jnp.maximum(prev, m)

    @pl.when(p == 1)
    def _phase_cast():
        for k, (w_ref, q_ref, s_ref) in enumerate((
                (w1_ref, q1_ref, s1_ref),
                (w2_ref, q2_ref, s2_ref),
                (w3_ref, q3_ref, s3_ref))):
            scale = jnp.maximum(msc_ref[k] / FP8_MAX, 1e-12)
            inv = 1.0 / scale
            q_ref[...] = jnp.clip(
                w_ref[...] * inv, -FP8_MAX, FP8_MAX).astype(jnp.float8_e4m3fn)
            s_ref[...] = scale.reshape(1, 1)


def _wprep(W1, W2, W3):
    wspec = pl.BlockSpec((WB, H), lambda p, i: (i, 0))
    qspec = pl.BlockSpec((WB, H), lambda p, i: (p * i, 0))
    sspec = pl.BlockSpec((1, 1), lambda p, i: (0, 0))
    return pl.pallas_call(
        _wprep_body,
        grid=(2, NB),
        in_specs=[wspec, wspec, wspec],
        out_specs=[qspec, qspec, qspec, sspec, sspec, sspec],
        out_shape=[jax.ShapeDtypeStruct((H, H), jnp.float8_e4m3fn)] * 3
        + [jax.ShapeDtypeStruct((1, 1), jnp.float32)] * 3,
        scratch_shapes=[pltpu.SMEM((3,), jnp.float32)],
        compiler_params=pltpu.CompilerParams(
            dimension_semantics=("arbitrary", "arbitrary"),
        ),
        name="wprep",
    )(W1, W2, W3)


def _layer_body(do_relu, do_final_norm, *refs):
    resid_ref, nw_ref, qw_ref, sw_ref, out_ref = refs
    r = resid_ref[...]
    if do_relu:
        r = jnp.maximum(r, 0.0)
    t = r * nw_ref[0:1, :]
    var = jnp.mean(r * r, axis=-1, keepdims=True)
    rs = jax.lax.rsqrt(var + EPS)
    amax = rs * jnp.max(jnp.abs(t), axis=-1, keepdims=True)
    s = jnp.maximum(amax / FP8_MAX, 1e-12)
    q = jnp.clip(t * (rs / s), -FP8_MAX, FP8_MAX).astype(jnp.float8_e4m3fn)
    out_ref[...] = jax.lax.dot_general(
        q, qw_ref[...], (((1,), (1,)), ((), ())),
        preferred_element_type=jnp.float32)
    new_resid = out_ref[...] * (s * sw_ref[0, 0]) + r
    if do_final_norm:
        var2 = jnp.mean(new_resid * new_resid, axis=-1, keepdims=True)
        out_ref[...] = new_resid * jax.lax.rsqrt(var2 + EPS) * nw_ref[1:2, :]
    else:
        out_ref[...] = new_resid


def _layer(resid, nw, qw, sw, nwf=None, do_relu=False):
    nwarr = (nw.reshape(1, H) if nwf is None
             else jnp.stack([nw, nwf], axis=0))
    in_specs = [
        pl.BlockSpec((BM, H), lambda i: (i, 0)),
        pl.BlockSpec(nwarr.shape, lambda i: (0, 0)),
        pl.BlockSpec((H, H), lambda i: (0, 0)),
        pl.BlockSpec(memory_space=pltpu.SMEM),
    ]
    args = [resid, nwarr, qw, sw]
    body = functools.partial(_layer_body, do_relu, nwf is not None)
    return pl.pallas_call(
        body,
        grid=(N_TOK // BM,),
        in_specs=in_specs,
        out_specs=pl.BlockSpec((BM, H), lambda i: (i, 0)),
        out_shape=jax.ShapeDtypeStruct((N_TOK, H), jnp.float32),
        compiler_params=pltpu.CompilerParams(
            dimension_semantics=("parallel",),
            vmem_limit_bytes=58 * 1024 * 1024,
        ),
        name="fused_layer",
    )(*args)


FBM = 256


def _fused2_body(x_ref, nw0_ref, nw1_ref, qw1_ref, qw2_ref, s_ref, out_ref):
    r = jnp.maximum(x_ref[...], 0.0)
    for li, (nw_ref, qw_ref) in enumerate(
            ((nw0_ref, qw1_ref), (nw1_ref, qw2_ref))):
        t = r * nw_ref[...]
        var = jnp.mean(r * r, axis=-1, keepdims=True)
        rs = jax.lax.rsqrt(var + EPS)
        amax = rs * jnp.max(jnp.abs(t), axis=-1, keepdims=True)
        s = jnp.maximum(amax / FP8_MAX, 1e-12)
        q = jnp.clip(t * (rs / s), -FP8_MAX, FP8_MAX).astype(
            jnp.float8_e4m3fn)
        acc = jax.lax.dot_general(
            q, qw_ref[...], (((1,), (1,)), ((), ())),
            preferred_element_type=jnp.float32)
        r = acc * (s * s_ref[0, li]) + r
    out_ref[...] = r


def _fused2(x, nw0, nw1, qw1, qw2, sws):
    vspec = pl.BlockSpec((1, H), lambda i: (0, 0))
    wspec = pl.BlockSpec((H, H), lambda i: (0, 0))
    return pl.pallas_call(
        _fused2_body,
        grid=(N_TOK // FBM,),
        in_specs=[
            pl.BlockSpec((FBM, H), lambda i: (i, 0)),
            vspec, vspec,
            wspec, wspec,
            pl.BlockSpec(memory_space=pltpu.SMEM),
        ],
        out_specs=pl.BlockSpec((FBM, H), lambda i: (i, 0)),
        out_shape=jax.ShapeDtypeStruct((N_TOK, H), jnp.float32),
        compiler_params=pltpu.CompilerParams(
            dimension_semantics=("parallel",),
            vmem_limit_bytes=58 * 1024 * 1024,
        ),
        name="fused2",
    )(x, nw0.reshape(1, H), nw1.reshape(1, H), qw1, qw2, sws)


def kernel(x, nw0, nw1, nw2, nw3, W1, W2, W3):
    qw1, qw2, qw3, s1, s2, s3 = _wprep(W1, W2, W3)
    sws = jnp.concatenate([s1, s2], axis=1)
    h2 = _fused2(x, nw0, nw1, qw1, qw2, sws)
    return _layer(h2, nw2, qw3, s3, nwf=nw3)
